# baseline (speedup 1.0000x reference)
import jax
import jax.numpy as jnp
from jax import lax
from jax.experimental import pallas as pl
from jax.experimental.pallas import tpu as pltpu
from jax.experimental.pallas import tpu_sc as plsc

N_EXP = 8
H = 1024
I = 4096
T = 2048
BT = 256
NB = T // BT + N_EXP
P = NB * BT
KC = 2048
NK = I // KC

try:
    _sc_info = plsc.get_sparse_core_info()
    _SC_NC, _SC_NS = _sc_info.num_cores, _sc_info.num_subcores
except Exception:
    _SC_NC, _SC_NS = 2, 16
NW = _SC_NC * _SC_NS
RPW = P // NW
NCH = 4
CH = RPW // NCH


def _silu(h):
    return h / (1.0 + jnp.exp(-h))


def _pack_bf16(y):
    lo = lax.bitcast_convert_type(
        y[:, :H // 2].astype(jnp.bfloat16), jnp.uint16).astype(jnp.uint32)
    hi = lax.bitcast_convert_type(
        y[:, H // 2:].astype(jnp.bfloat16), jnp.uint16).astype(jnp.uint32)
    return lax.bitcast_convert_type((hi << 16) | lo, jnp.int32)


def _unpack_bf16(p):
    v = lax.bitcast_convert_type(p, jnp.uint32)
    lo = lax.bitcast_convert_type(
        (v & 0xFFFF).astype(jnp.uint16), jnp.bfloat16).astype(jnp.float32)
    hi = lax.bitcast_convert_type(
        (v >> 16).astype(jnp.uint16), jnp.bfloat16).astype(jnp.float32)
    return jnp.concatenate([lo, hi], axis=-1)


def _route_body(x_ref, rw_ref, src_ref, dst_ref, meta_ref, xp_ref):
    logits = jnp.dot(x_ref[...], rw_ref[...].T,
                     preferred_element_type=jnp.float32)
    ii = lax.broadcasted_iota(jnp.int32, (T, N_EXP), 1)
    m1 = jnp.max(logits, axis=-1, keepdims=True)
    i1 = jnp.min(jnp.where(logits == m1, ii, N_EXP), axis=-1, keepdims=True)
    l2 = jnp.where(ii == i1, -jnp.inf, logits)
    m2 = jnp.max(l2, axis=-1, keepdims=True)
    i2 = jnp.min(jnp.where(l2 == m2, ii, N_EXP), axis=-1, keepdims=True)
    etok = jnp.maximum(i1, i2)

    oh = (etok == ii).astype(jnp.float32)
    counts = jnp.sum(oh, axis=0, keepdims=True)
    cum = oh
    sh = 1
    while sh < T:
        cum = cum + jnp.concatenate(
            [jnp.zeros((sh, N_EXP), jnp.float32), cum[:T - sh]], axis=0)
        sh *= 2
    rank = jnp.sum(oh * cum, axis=-1, keepdims=True) - 1.0

    pc = jnp.floor((counts + (BT - 1)) / BT) * BT
    ei = lax.broadcasted_iota(jnp.int32, (N_EXP, N_EXP), 0)
    ej = lax.broadcasted_iota(jnp.int32, (N_EXP, N_EXP), 1)
    pcb = jnp.broadcast_to(pc, (N_EXP, N_EXP))
    pstart = jnp.sum(jnp.where(ej < ei, pcb, 0.0), axis=-1,
                     keepdims=True)
    pend = pstart + pc.reshape(N_EXP, 1)

    slot_base = jnp.sum(oh * pstart.reshape(1, N_EXP), axis=-1,
                        keepdims=True)
    slot = (slot_base + rank).astype(jnp.int32)

    tok = lax.broadcasted_iota(jnp.int32, (T, BT), 0)
    for pb in range(NB):
        pvec = (lax.broadcasted_iota(jnp.int32, (1, BT), 1) + pb * BT)
        match = slot == pvec
        srcs = jnp.sum(jnp.where(match, tok, 0), axis=0, keepdims=True)
        valid = jnp.sum(match.astype(jnp.int32), axis=0, keepdims=True)
        src_ref[pb:pb + 1, :] = jnp.where(valid > 0, srcs, 0)
        dst_ref[pb:pb + 1, :] = jnp.where(valid > 0, srcs, T)

    has = (counts.reshape(N_EXP, 1) > 0).astype(jnp.int32)
    maxe = jnp.max(has * ei[:, :1], axis=0, keepdims=True)
    bidx = lax.broadcasted_iota(jnp.int32, (1, BT), 1) * BT
    owner = jnp.sum((pend <= bidx.astype(jnp.float32)).astype(jnp.int32),
                    axis=0, keepdims=True)
    owner = jnp.minimum(owner, maxe)
    active = (bidx < pend[N_EXP - 1].reshape(1, 1)).astype(jnp.int32)
    meta_ref[0:1, :] = owner
    meta_ref[1:2, :] = active
    xp_ref[...] = _pack_bf16(x_ref[...])


def _route(x2, router_w):
    return pl.pallas_call(
        _route_body,
        in_specs=[
            pl.BlockSpec((T, H), lambda: (0, 0)),
            pl.BlockSpec((N_EXP, H), lambda: (0, 0)),
        ],
        out_specs=[
            pl.BlockSpec((NB, BT), lambda: (0, 0)),
            pl.BlockSpec((NB, BT), lambda: (0, 0)),
            pl.BlockSpec((2, BT), lambda: (0, 0)),
            pl.BlockSpec((T, H // 2), lambda: (0, 0)),
        ],
        out_shape=[
            jax.ShapeDtypeStruct((NB, BT), jnp.int32),
            jax.ShapeDtypeStruct((NB, BT), jnp.int32),
            jax.ShapeDtypeStruct((2, BT), jnp.int32),
            jax.ShapeDtypeStruct((T, H // 2), jnp.int32),
        ],
    )(x2, router_w)


def _sc_mesh():
    return plsc.VectorSubcoreMesh(core_axis_name="c", subcore_axis_name="s")


def _sc_wid():
    return lax.axis_index("s") * _SC_NC + lax.axis_index("c")


def _gather(x2, src2d):
    def body(x_hbm, src_hbm, out_hbm, idx_v, rows_v, gsem, ssem):
        wid = _sc_wid()
        base = wid * RPW
        pltpu.sync_copy(src_hbm.at[pl.ds(wid * NCH, NCH)], idx_v)
        gets = [
            pltpu.async_copy(x_hbm.at[idx_v.at[c]], rows_v.at[c], gsem)
            for c in range(NCH)
        ]
        puts = []
        for c in range(NCH):
            gets[c].wait()
            puts.append(pltpu.async_copy(
                rows_v.at[c], out_hbm.at[pl.ds(base + c * CH, CH)], ssem))
        for p in puts:
            p.wait()

    return pl.kernel(
        body,
        mesh=_sc_mesh(),
        out_type=jax.ShapeDtypeStruct((P, H // 2), jnp.int32),
        scratch_types=[
            pltpu.VMEM((NCH, CH), jnp.int32),
            pltpu.VMEM((NCH, CH, H // 2), jnp.int32),
            pltpu.SemaphoreType.DMA,
            pltpu.SemaphoreType.DMA,
        ],
    )(x2, src2d)


def _scatter(yg, dst2d):
    def body(y_hbm, dst_hbm, out_hbm, idx_v, rows_v, gsem, ssem):
        wid = _sc_wid()
        base = wid * RPW
        pltpu.sync_copy(dst_hbm.at[pl.ds(wid * NCH, NCH)], idx_v)
        gets = [
            pltpu.async_copy(
                y_hbm.at[pl.ds(base + c * CH, CH)], rows_v.at[c], gsem)
            for c in range(NCH)
        ]
        puts = []
        for c in range(NCH):
            gets[c].wait()
            puts.append(pltpu.async_copy(
                rows_v.at[c], out_hbm.at[idx_v.at[c]], ssem))
        for p in puts:
            p.wait()

    return pl.kernel(
        body,
        mesh=_sc_mesh(),
        out_type=jax.ShapeDtypeStruct((T + 1, H // 2), jnp.int32),
        scratch_types=[
            pltpu.VMEM((NCH, CH), jnp.int32),
            pltpu.VMEM((NCH, CH, H // 2), jnp.int32),
            pltpu.SemaphoreType.DMA,
            pltpu.SemaphoreType.DMA,
        ],
    )(yg, dst2d)


def _ffn_body(meta_ref, xg_ref, w1_ref, w2_ref, o_ref, accs):
    k = pl.program_id(0)
    b = pl.program_id(1)

    @pl.when(meta_ref[1, b] == 1)
    def _():
        h = _silu(jnp.dot(_unpack_bf16(xg_ref[...]), w1_ref[0].T,
                          preferred_element_type=jnp.float32))
        part = jnp.dot(h, w2_ref[0].T, preferred_element_type=jnp.float32)
        if NK == 1:
            o_ref[...] = _pack_bf16(part)
        else:
            @pl.when(k == 0)
            def _():
                accs[b] = part

            @pl.when(jnp.logical_and(k > 0, k < NK - 1))
            def _():
                accs[b] = accs[b] + part

            @pl.when(k == NK - 1)
            def _():
                o_ref[...] = _pack_bf16(accs[b] + part)


def _ffn(xg, w1, w2, meta):
    grid_spec = pltpu.PrefetchScalarGridSpec(
        num_scalar_prefetch=1,
        grid=(NK, NB),
        in_specs=[
            pl.BlockSpec((BT, H // 2), lambda k, b, m: (b, 0)),
            pl.BlockSpec((1, KC, H), lambda k, b, m: (m[0, b], k, 0)),
            pl.BlockSpec((1, H, KC), lambda k, b, m: (m[0, b], 0, k)),
        ],
        out_specs=pl.BlockSpec((BT, H // 2), lambda k, b, m: (jnp.where(k == NK - 1, b, 0), 0)),
        scratch_shapes=[pltpu.VMEM((NB, BT, H), jnp.float32)],
    )
    return pl.pallas_call(
        _ffn_body,
        grid_spec=grid_spec,
        out_shape=jax.ShapeDtypeStruct((P, H // 2), jnp.int32),
    )(meta, xg, w1, w2)


def kernel(x, router_w, w1, w2):
    b, s, h = x.shape
    x2 = x.reshape(-1, h)
    src, dst, meta, xp = _route(x2, router_w)
    xg = _gather(xp, src.reshape(NW * NCH, CH))
    yg = _ffn(xg, w1, w2, meta)
    out_pad = _scatter(yg, dst.reshape(NW * NCH, CH))
    return _unpack_bf16(out_pad[:T]).reshape(b, s, h)


# speedup vs baseline: 1.0046x; 1.0046x over previous
import jax
import jax.numpy as jnp
from jax import lax
from jax.experimental import pallas as pl
from jax.experimental.pallas import tpu as pltpu
from jax.experimental.pallas import tpu_sc as plsc

N_EXP = 8
H = 1024
I = 4096
T = 2048
BT = 128
NB = T // BT + N_EXP
P = NB * BT
KC = 1024
NK = I // KC

try:
    _sc_info = plsc.get_sparse_core_info()
    _SC_NC, _SC_NS = _sc_info.num_cores, _sc_info.num_subcores
except Exception:
    _SC_NC, _SC_NS = 2, 16
NW = _SC_NC * _SC_NS
RPW = P // NW
NCH = 4
CH = RPW // NCH


def _silu(h):
    return h / (1.0 + jnp.exp(-h))


def _pack_bf16(y):
    lo = lax.bitcast_convert_type(
        y[:, :H // 2].astype(jnp.bfloat16), jnp.uint16).astype(jnp.uint32)
    hi = lax.bitcast_convert_type(
        y[:, H // 2:].astype(jnp.bfloat16), jnp.uint16).astype(jnp.uint32)
    return lax.bitcast_convert_type((hi << 16) | lo, jnp.int32)


def _unpack_bf16(p):
    v = lax.bitcast_convert_type(p, jnp.uint32)
    lo = lax.bitcast_convert_type(
        (v & 0xFFFF).astype(jnp.uint16), jnp.bfloat16).astype(jnp.float32)
    hi = lax.bitcast_convert_type(
        (v >> 16).astype(jnp.uint16), jnp.bfloat16).astype(jnp.float32)
    return jnp.concatenate([lo, hi], axis=-1)


def _route_body(x_ref, rw_ref, src_ref, dst_ref, meta_ref, xp_ref):
    logits = jnp.dot(x_ref[...], rw_ref[...].T,
                     preferred_element_type=jnp.float32)
    ii = lax.broadcasted_iota(jnp.int32, (T, N_EXP), 1)
    m1 = jnp.max(logits, axis=-1, keepdims=True)
    i1 = jnp.min(jnp.where(logits == m1, ii, N_EXP), axis=-1, keepdims=True)
    l2 = jnp.where(ii == i1, -jnp.inf, logits)
    m2 = jnp.max(l2, axis=-1, keepdims=True)
    i2 = jnp.min(jnp.where(l2 == m2, ii, N_EXP), axis=-1, keepdims=True)
    etok = jnp.maximum(i1, i2)

    oh = (etok == ii).astype(jnp.float32)
    counts = jnp.sum(oh, axis=0, keepdims=True)
    cum = oh
    sh = 1
    while sh < T:
        cum = cum + jnp.concatenate(
            [jnp.zeros((sh, N_EXP), jnp.float32), cum[:T - sh]], axis=0)
        sh *= 2
    rank = jnp.sum(oh * cum, axis=-1, keepdims=True) - 1.0

    pc = jnp.floor((counts + (BT - 1)) / BT) * BT
    ei = lax.broadcasted_iota(jnp.int32, (N_EXP, N_EXP), 0)
    ej = lax.broadcasted_iota(jnp.int32, (N_EXP, N_EXP), 1)
    pcb = jnp.broadcast_to(pc, (N_EXP, N_EXP))
    pstart = jnp.sum(jnp.where(ej < ei, pcb, 0.0), axis=-1,
                     keepdims=True)
    pend = pstart + pc.reshape(N_EXP, 1)

    slot_base = jnp.sum(oh * pstart.reshape(1, N_EXP), axis=-1,
                        keepdims=True)
    slot = (slot_base + rank).astype(jnp.int32)

    tok = lax.broadcasted_iota(jnp.int32, (T, BT), 0)
    for pb in range(NB):
        pvec = (lax.broadcasted_iota(jnp.int32, (1, BT), 1) + pb * BT)
        match = slot == pvec
        srcs = jnp.sum(jnp.where(match, tok, 0), axis=0, keepdims=True)
        valid = jnp.sum(match.astype(jnp.int32), axis=0, keepdims=True)
        src_ref[pb:pb + 1, :] = jnp.where(valid > 0, srcs, 0)
        dst_ref[pb:pb + 1, :] = jnp.where(valid > 0, srcs, T)

    has = (counts.reshape(N_EXP, 1) > 0).astype(jnp.int32)
    maxe = jnp.max(has * ei[:, :1], axis=0, keepdims=True)
    bidx = lax.broadcasted_iota(jnp.int32, (1, BT), 1) * BT
    owner = jnp.sum((pend <= bidx.astype(jnp.float32)).astype(jnp.int32),
                    axis=0, keepdims=True)
    owner = jnp.minimum(owner, maxe)
    active = (bidx < pend[N_EXP - 1].reshape(1, 1)).astype(jnp.int32)
    meta_ref[0:1, :] = owner
    meta_ref[1:2, :] = active
    xp_ref[...] = _pack_bf16(x_ref[...])


def _route(x2, router_w):
    return pl.pallas_call(
        _route_body,
        in_specs=[
            pl.BlockSpec((T, H), lambda: (0, 0)),
            pl.BlockSpec((N_EXP, H), lambda: (0, 0)),
        ],
        out_specs=[
            pl.BlockSpec((NB, BT), lambda: (0, 0)),
            pl.BlockSpec((NB, BT), lambda: (0, 0)),
            pl.BlockSpec((2, BT), lambda: (0, 0)),
            pl.BlockSpec((T, H // 2), lambda: (0, 0)),
        ],
        out_shape=[
            jax.ShapeDtypeStruct((NB, BT), jnp.int32),
            jax.ShapeDtypeStruct((NB, BT), jnp.int32),
            jax.ShapeDtypeStruct((2, BT), jnp.int32),
            jax.ShapeDtypeStruct((T, H // 2), jnp.int32),
        ],
    )(x2, router_w)


def _sc_mesh():
    return plsc.VectorSubcoreMesh(core_axis_name="c", subcore_axis_name="s")


def _sc_wid():
    return lax.axis_index("s") * _SC_NC + lax.axis_index("c")


def _gather(x2, src2d):
    def body(x_hbm, src_hbm, out_hbm, idx_v, rows_v, gsem, ssem):
        wid = _sc_wid()
        base = wid * RPW
        pltpu.sync_copy(src_hbm.at[pl.ds(wid * NCH, NCH)], idx_v)
        gets = [
            pltpu.async_copy(x_hbm.at[idx_v.at[c]], rows_v.at[c], gsem)
            for c in range(NCH)
        ]
        puts = []
        for c in range(NCH):
            gets[c].wait()
            puts.append(pltpu.async_copy(
                rows_v.at[c], out_hbm.at[pl.ds(base + c * CH, CH)], ssem))
        for p in puts:
            p.wait()

    return pl.kernel(
        body,
        mesh=_sc_mesh(),
        out_type=jax.ShapeDtypeStruct((P, H // 2), jnp.int32),
        scratch_types=[
            pltpu.VMEM((NCH, CH), jnp.int32),
            pltpu.VMEM((NCH, CH, H // 2), jnp.int32),
            pltpu.SemaphoreType.DMA,
            pltpu.SemaphoreType.DMA,
        ],
    )(x2, src2d)


def _scatter(yg, dst2d):
    def body(y_hbm, dst_hbm, out_hbm, idx_v, rows_v, gsem, ssem):
        wid = _sc_wid()
        base = wid * RPW
        pltpu.sync_copy(dst_hbm.at[pl.ds(wid * NCH, NCH)], idx_v)
        gets = [
            pltpu.async_copy(
                y_hbm.at[pl.ds(base + c * CH, CH)], rows_v.at[c], gsem)
            for c in range(NCH)
        ]
        puts = []
        for c in range(NCH):
            gets[c].wait()
            puts.append(pltpu.async_copy(
                rows_v.at[c], out_hbm.at[idx_v.at[c]], ssem))
        for p in puts:
            p.wait()

    return pl.kernel(
        body,
        mesh=_sc_mesh(),
        out_type=jax.ShapeDtypeStruct((T + 1, H // 2), jnp.int32),
        scratch_types=[
            pltpu.VMEM((NCH, CH), jnp.int32),
            pltpu.VMEM((NCH, CH, H // 2), jnp.int32),
            pltpu.SemaphoreType.DMA,
            pltpu.SemaphoreType.DMA,
        ],
    )(yg, dst2d)


def _ffn_body(meta_ref, xg_ref, w1_ref, w2_ref, o_ref, accs):
    k = pl.program_id(0)
    b = pl.program_id(1)

    @pl.when(meta_ref[1, b] == 1)
    def _():
        h = _silu(jnp.dot(_unpack_bf16(xg_ref[...]), w1_ref[0].T,
                          preferred_element_type=jnp.float32))
        part = jnp.dot(h, w2_ref[0].T, preferred_element_type=jnp.float32)
        if NK == 1:
            o_ref[...] = _pack_bf16(part)
        else:
            @pl.when(k == 0)
            def _():
                accs[b] = part

            @pl.when(jnp.logical_and(k > 0, k < NK - 1))
            def _():
                accs[b] = accs[b] + part

            @pl.when(k == NK - 1)
            def _():
                o_ref[...] = _pack_bf16(accs[b] + part)


def _ffn(xg, w1, w2, meta):
    grid_spec = pltpu.PrefetchScalarGridSpec(
        num_scalar_prefetch=1,
        grid=(NK, NB),
        in_specs=[
            pl.BlockSpec((BT, H // 2), lambda k, b, m: (b, 0)),
            pl.BlockSpec((1, KC, H), lambda k, b, m: (m[0, b], k, 0)),
            pl.BlockSpec((1, H, KC), lambda k, b, m: (m[0, b], 0, k)),
        ],
        out_specs=pl.BlockSpec((BT, H // 2), lambda k, b, m: (jnp.where(k == NK - 1, b, 0), 0)),
        scratch_shapes=[pltpu.VMEM((NB, BT, H), jnp.float32)],
    )
    return pl.pallas_call(
        _ffn_body,
        grid_spec=grid_spec,
        out_shape=jax.ShapeDtypeStruct((P, H // 2), jnp.int32),
    )(meta, xg, w1, w2)


def kernel(x, router_w, w1, w2):
    b, s, h = x.shape
    x2 = x.reshape(-1, h)
    src, dst, meta, xp = _route(x2, router_w)
    xg = _gather(xp, src.reshape(NW * NCH, CH))
    yg = _ffn(xg, w1, w2, meta)
    out_pad = _scatter(yg, dst.reshape(NW * NCH, CH))
    return _unpack_bf16(out_pad[:T]).reshape(b, s, h)


# speedup vs baseline: 1.0894x; 1.0844x over previous
import jax
import jax.numpy as jnp
from jax import lax
from jax.experimental import pallas as pl
from jax.experimental.pallas import tpu as pltpu
from jax.experimental.pallas import tpu_sc as plsc

N_EXP = 8
H = 1024
I = 4096
T = 2048
BT = 128
NB = T // BT + N_EXP
P = NB * BT
KC = 2048
NK = I // KC

try:
    _sc_info = plsc.get_sparse_core_info()
    _SC_NC, _SC_NS = _sc_info.num_cores, _sc_info.num_subcores
except Exception:
    _SC_NC, _SC_NS = 2, 16
NW = _SC_NC * _SC_NS
RPW = P // NW
NCH = 4
CH = RPW // NCH


def _silu(h):
    return h / (1.0 + jnp.exp(-h))


def _pack_bf16(y):
    lo = lax.bitcast_convert_type(
        y[:, :H // 2].astype(jnp.bfloat16), jnp.uint16).astype(jnp.uint32)
    hi = lax.bitcast_convert_type(
        y[:, H // 2:].astype(jnp.bfloat16), jnp.uint16).astype(jnp.uint32)
    return lax.bitcast_convert_type((hi << 16) | lo, jnp.int32)


def _unpack_bf16(p):
    v = lax.bitcast_convert_type(p, jnp.uint32)
    lo = lax.bitcast_convert_type(
        (v & 0xFFFF).astype(jnp.uint16), jnp.bfloat16).astype(jnp.float32)
    hi = lax.bitcast_convert_type(
        (v >> 16).astype(jnp.uint16), jnp.bfloat16).astype(jnp.float32)
    return jnp.concatenate([lo, hi], axis=-1)


def _route_body(x_ref, rw_ref, src_ref, dst_ref, meta_ref, xp_ref):
    logits = jnp.dot(x_ref[...], rw_ref[...].T,
                     preferred_element_type=jnp.float32)
    ii = lax.broadcasted_iota(jnp.int32, (T, N_EXP), 1)
    m1 = jnp.max(logits, axis=-1, keepdims=True)
    i1 = jnp.min(jnp.where(logits == m1, ii, N_EXP), axis=-1, keepdims=True)
    l2 = jnp.where(ii == i1, -jnp.inf, logits)
    m2 = jnp.max(l2, axis=-1, keepdims=True)
    i2 = jnp.min(jnp.where(l2 == m2, ii, N_EXP), axis=-1, keepdims=True)
    etok = jnp.maximum(i1, i2)

    oh = (etok == ii).astype(jnp.float32)
    counts = jnp.sum(oh, axis=0, keepdims=True)
    cum = oh
    sh = 1
    while sh < T:
        cum = cum + jnp.concatenate(
            [jnp.zeros((sh, N_EXP), jnp.float32), cum[:T - sh]], axis=0)
        sh *= 2
    rank = jnp.sum(oh * cum, axis=-1, keepdims=True) - 1.0

    pc = jnp.floor((counts + (BT - 1)) / BT) * BT
    ei = lax.broadcasted_iota(jnp.int32, (N_EXP, N_EXP), 0)
    ej = lax.broadcasted_iota(jnp.int32, (N_EXP, N_EXP), 1)
    pcb = jnp.broadcast_to(pc, (N_EXP, N_EXP))
    pstart = jnp.sum(jnp.where(ej < ei, pcb, 0.0), axis=-1,
                     keepdims=True)
    pend = pstart + pc.reshape(N_EXP, 1)

    slot_base = jnp.sum(oh * pstart.reshape(1, N_EXP), axis=-1,
                        keepdims=True)
    slot = (slot_base + rank).astype(jnp.int32)

    tok = lax.broadcasted_iota(jnp.int32, (T, BT), 0)
    for pb in range(NB):
        pvec = (lax.broadcasted_iota(jnp.int32, (1, BT), 1) + pb * BT)
        match = slot == pvec
        srcs = jnp.sum(jnp.where(match, tok, 0), axis=0, keepdims=True)
        valid = jnp.sum(match.astype(jnp.int32), axis=0, keepdims=True)
        src_ref[pb:pb + 1, :] = jnp.where(valid > 0, srcs, 0)
        dst_ref[pb:pb + 1, :] = jnp.where(valid > 0, srcs, T)

    has = (counts.reshape(N_EXP, 1) > 0).astype(jnp.int32)
    maxe = jnp.max(has * ei[:, :1], axis=0, keepdims=True)
    bidx = lax.broadcasted_iota(jnp.int32, (1, BT), 1) * BT
    owner = jnp.sum((pend <= bidx.astype(jnp.float32)).astype(jnp.int32),
                    axis=0, keepdims=True)
    owner = jnp.minimum(owner, maxe)
    active = (bidx < pend[N_EXP - 1].reshape(1, 1)).astype(jnp.int32)
    meta_ref[0:1, :] = owner
    meta_ref[1:2, :] = active
    xp_ref[...] = _pack_bf16(x_ref[...])


def _route(x2, router_w):
    return pl.pallas_call(
        _route_body,
        in_specs=[
            pl.BlockSpec((T, H), lambda: (0, 0)),
            pl.BlockSpec((N_EXP, H), lambda: (0, 0)),
        ],
        out_specs=[
            pl.BlockSpec((NB, BT), lambda: (0, 0)),
            pl.BlockSpec((NB, BT), lambda: (0, 0)),
            pl.BlockSpec((2, BT), lambda: (0, 0)),
            pl.BlockSpec((T, H // 2), lambda: (0, 0)),
        ],
        out_shape=[
            jax.ShapeDtypeStruct((NB, BT), jnp.int32),
            jax.ShapeDtypeStruct((NB, BT), jnp.int32),
            jax.ShapeDtypeStruct((2, BT), jnp.int32),
            jax.ShapeDtypeStruct((T, H // 2), jnp.int32),
        ],
    )(x2, router_w)


def _sc_mesh():
    return plsc.VectorSubcoreMesh(core_axis_name="c", subcore_axis_name="s")


def _sc_wid():
    return lax.axis_index("s") * _SC_NC + lax.axis_index("c")


def _gather(x2, src2d):
    def body(x_hbm, src_hbm, out_hbm, idx_v, rows_v, gsem, ssem):
        wid = _sc_wid()
        base = wid * RPW
        pltpu.sync_copy(src_hbm.at[pl.ds(wid * NCH, NCH)], idx_v)
        gets = [
            pltpu.async_copy(x_hbm.at[idx_v.at[c]], rows_v.at[c], gsem)
            for c in range(NCH)
        ]
        puts = []
        for c in range(NCH):
            gets[c].wait()
            puts.append(pltpu.async_copy(
                rows_v.at[c], out_hbm.at[pl.ds(base + c * CH, CH)], ssem))
        for p in puts:
            p.wait()

    return pl.kernel(
        body,
        mesh=_sc_mesh(),
        out_type=jax.ShapeDtypeStruct((P, H // 2), jnp.int32),
        scratch_types=[
            pltpu.VMEM((NCH, CH), jnp.int32),
            pltpu.VMEM((NCH, CH, H // 2), jnp.int32),
            pltpu.SemaphoreType.DMA,
            pltpu.SemaphoreType.DMA,
        ],
    )(x2, src2d)


def _scatter(yg, dst2d):
    def body(y_hbm, dst_hbm, out_hbm, idx_v, rows_v, gsem, ssem):
        wid = _sc_wid()
        base = wid * RPW
        pltpu.sync_copy(dst_hbm.at[pl.ds(wid * NCH, NCH)], idx_v)
        gets = [
            pltpu.async_copy(
                y_hbm.at[pl.ds(base + c * CH, CH)], rows_v.at[c], gsem)
            for c in range(NCH)
        ]
        puts = []
        for c in range(NCH):
            gets[c].wait()
            puts.append(pltpu.async_copy(
                rows_v.at[c], out_hbm.at[idx_v.at[c]], ssem))
        for p in puts:
            p.wait()

    return pl.kernel(
        body,
        mesh=_sc_mesh(),
        out_type=jax.ShapeDtypeStruct((T + 1, H // 2), jnp.int32),
        scratch_types=[
            pltpu.VMEM((NCH, CH), jnp.int32),
            pltpu.VMEM((NCH, CH, H // 2), jnp.int32),
            pltpu.SemaphoreType.DMA,
            pltpu.SemaphoreType.DMA,
        ],
    )(yg, dst2d)


def _ffn_body(meta_ref, xg_ref, w1_ref, w2_ref, o_ref, accs):
    k = pl.program_id(0)
    b = pl.program_id(1)

    @pl.when(meta_ref[1, b] == 1)
    def _():
        h = _silu(jnp.dot(_unpack_bf16(xg_ref[...]), w1_ref[0].T,
                          preferred_element_type=jnp.float32))
        part = jnp.dot(h, w2_ref[0].T, preferred_element_type=jnp.float32)
        if NK == 1:
            o_ref[...] = _pack_bf16(part)
        else:
            @pl.when(k == 0)
            def _():
                accs[b] = part

            @pl.when(jnp.logical_and(k > 0, k < NK - 1))
            def _():
                accs[b] = accs[b] + part

            @pl.when(k == NK - 1)
            def _():
                o_ref[...] = _pack_bf16(accs[b] + part)


def _ffn(xg, w1, w2, meta):
    grid_spec = pltpu.PrefetchScalarGridSpec(
        num_scalar_prefetch=1,
        grid=(NK, NB),
        in_specs=[
            pl.BlockSpec((BT, H // 2), lambda k, b, m: (b, 0)),
            pl.BlockSpec((1, KC, H), lambda k, b, m: (m[0, b], k, 0)),
            pl.BlockSpec((1, H, KC), lambda k, b, m: (m[0, b], 0, k)),
        ],
        out_specs=pl.BlockSpec((BT, H // 2), lambda k, b, m: (jnp.where(k == NK - 1, b, 0), 0)),
        scratch_shapes=[pltpu.VMEM((NB, BT, H), jnp.float32)],
    )
    return pl.pallas_call(
        _ffn_body,
        grid_spec=grid_spec,
        out_shape=jax.ShapeDtypeStruct((P, H // 2), jnp.int32),
    )(meta, xg, w1, w2)


def kernel(x, router_w, w1, w2):
    b, s, h = x.shape
    x2 = x.reshape(-1, h)
    src, dst, meta, xp = _route(x2, router_w)
    xg = _gather(xp, src.reshape(NW * NCH, CH))
    yg = _ffn(xg, w1, w2, meta)
    out_pad = _scatter(yg, dst.reshape(NW * NCH, CH))
    return _unpack_bf16(out_pad[:T]).reshape(b, s, h)
